# Initial kernel scaffold; baseline (speedup 1.0000x reference)
import jax
import jax.numpy as jnp
from jax.experimental import pallas as pl


def kernel(x):
    raise NotImplementedError("write your pallas kernel here")


# speedup vs baseline: 5.4308x; 5.4308x over previous
import functools

import jax
import jax.numpy as jnp
from jax import lax
from jax.experimental import pallas as pl
from jax.experimental.pallas import tpu as pltpu
from jax.experimental.pallas import tpu_sc as plsc

N_NEG = 10

_NC = 2
_NS = 16
_NW = _NC * _NS
_CHUNK = 128


@functools.lru_cache(maxsize=None)
def _make_gather(n_neg_rows, n_tgt_rows, C):
    neg_per = n_neg_rows // _NW
    tgt_per = n_tgt_rows // _NW
    assert neg_per % _CHUNK == 0 and tgt_per % _CHUNK == 0

    mesh = plsc.VectorSubcoreMesh(core_axis_name="c", subcore_axis_name="s")

    @functools.partial(
        pl.kernel,
        mesh=mesh,
        out_type=[
            jax.ShapeDtypeStruct((n_neg_rows, C), jnp.float32),
            jax.ShapeDtypeStruct((n_tgt_rows, C), jnp.float32),
        ],
        scratch_types=[
            pltpu.VMEM((neg_per,), jnp.int32),
            pltpu.VMEM((tgt_per,), jnp.int32),
            pltpu.VMEM((_CHUNK, C), jnp.float32),
            pltpu.SemaphoreType.DMA,
        ],
    )
    def gather_k(table_hbm, negidx_hbm, tgtidx_hbm, negout_hbm, tgtout_hbm,
                 negidx_v, tgtidx_v, rows_v, sem):
        wid = lax.axis_index("s") * _NC + lax.axis_index("c")
        pltpu.sync_copy(negidx_hbm.at[pl.ds(wid * neg_per, neg_per)], negidx_v)
        pltpu.sync_copy(tgtidx_hbm.at[pl.ds(wid * tgt_per, tgt_per)], tgtidx_v)

        def neg_body(c, carry):
            base = c * _CHUNK
            pltpu.async_copy(
                table_hbm.at[negidx_v.at[pl.ds(base, _CHUNK)]], rows_v, sem
            ).wait()
            pltpu.sync_copy(
                rows_v, negout_hbm.at[pl.ds(wid * neg_per + base, _CHUNK)]
            )
            return carry

        lax.fori_loop(0, neg_per // _CHUNK, neg_body, 0)

        def tgt_body(c, carry):
            base = c * _CHUNK
            pltpu.async_copy(
                table_hbm.at[tgtidx_v.at[pl.ds(base, _CHUNK)]], rows_v, sem
            ).wait()
            pltpu.sync_copy(
                rows_v, tgtout_hbm.at[pl.ds(wid * tgt_per + base, _CHUNK)]
            )
            return carry

        lax.fori_loop(0, tgt_per // _CHUNK, tgt_body, 0)

    return gather_k


def kernel(x):
    B, T, C = x.shape
    N = N_NEG
    key = jax.random.key(42)
    raw = jax.random.randint(key, (B, N * T), 0, T - 1)
    tszs = jnp.repeat(jnp.arange(T), N)
    loc = jnp.where(raw >= tszs[None, :], raw + 1, raw)
    g = loc + 1
    g = jnp.where(g >= T, g - T, g)
    neg_idx = g + jnp.arange(B)[:, None] * T
    neg_idx = neg_idx.reshape(B, T, N).transpose(2, 0, 1).reshape(-1)
    neg_idx = neg_idx.astype(jnp.int32)

    t = jnp.arange(T)
    tshift = jnp.where(t + 1 >= T, 0, t + 1)
    tgt_idx = (tshift[None, :] + jnp.arange(B)[:, None] * T).reshape(-1)
    tgt_idx = tgt_idx.astype(jnp.int32)

    table = x.reshape(B * T, C)
    neg_out, tgt_out = _make_gather(N * B * T, B * T, C)(table, neg_idx, tgt_idx)
    return (x, tgt_out.reshape(B, T, C), neg_out.reshape(N, B, T, C))


# speedup vs baseline: 6.0228x; 1.1090x over previous
import functools

import jax
import jax.numpy as jnp
from jax import lax
from jax.experimental import pallas as pl
from jax.experimental.pallas import tpu as pltpu
from jax.experimental.pallas import tpu_sc as plsc

N_NEG = 10

_NC = 2
_NS = 16
_NW = _NC * _NS
_CHUNK = 128


@functools.lru_cache(maxsize=None)
def _make_gather(n_neg_rows, n_tgt_rows, C):
    neg_per = n_neg_rows // _NW
    tgt_per = n_tgt_rows // _NW
    assert neg_per % _CHUNK == 0 and tgt_per % _CHUNK == 0

    mesh = plsc.VectorSubcoreMesh(core_axis_name="c", subcore_axis_name="s")

    @functools.partial(
        pl.kernel,
        mesh=mesh,
        out_type=[
            jax.ShapeDtypeStruct((n_neg_rows, C), jnp.float32),
            jax.ShapeDtypeStruct((n_tgt_rows, C), jnp.float32),
        ],
        scratch_types=[
            pltpu.VMEM((neg_per,), jnp.int32),
            pltpu.VMEM((tgt_per,), jnp.int32),
            pltpu.VMEM((_CHUNK, C), jnp.float32),
            pltpu.VMEM((_CHUNK, C), jnp.float32),
            pltpu.SemaphoreType.DMA,
            pltpu.SemaphoreType.DMA,
            pltpu.SemaphoreType.DMA,
            pltpu.SemaphoreType.DMA,
        ],
    )
    def gather_k(table_hbm, negidx_hbm, tgtidx_hbm, negout_hbm, tgtout_hbm,
                 negidx_v, tgtidx_v, rows0_v, rows1_v,
                 sem_g0, sem_g1, sem_s0, sem_s1):
        wid = lax.axis_index("s") * _NC + lax.axis_index("c")
        pltpu.sync_copy(negidx_hbm.at[pl.ds(wid * neg_per, neg_per)], negidx_v)
        pltpu.sync_copy(tgtidx_hbm.at[pl.ds(wid * tgt_per, tgt_per)], tgtidx_v)

        def run_pipelined(idx_v, out_hbm, out_base, n_chunks):
            def drain(rows_v, sem):
                pltpu.make_async_copy(
                    rows_v, out_hbm.at[pl.ds(0, _CHUNK)], sem
                ).wait()

            def body(i, carry):
                c0 = 2 * i * _CHUNK
                c1 = c0 + _CHUNK

                @pl.when(i > 0)
                def _():
                    drain(rows0_v, sem_s0)
                    drain(rows1_v, sem_s1)

                g0 = pltpu.async_copy(
                    table_hbm.at[idx_v.at[pl.ds(c0, _CHUNK)]], rows0_v, sem_g0
                )
                g1 = pltpu.async_copy(
                    table_hbm.at[idx_v.at[pl.ds(c1, _CHUNK)]], rows1_v, sem_g1
                )
                g0.wait()
                pltpu.async_copy(
                    rows0_v, out_hbm.at[pl.ds(out_base + c0, _CHUNK)], sem_s0
                )
                g1.wait()
                pltpu.async_copy(
                    rows1_v, out_hbm.at[pl.ds(out_base + c1, _CHUNK)], sem_s1
                )
                return carry

            lax.fori_loop(0, n_chunks // 2, body, 0)
            drain(rows0_v, sem_s0)
            drain(rows1_v, sem_s1)

        run_pipelined(negidx_v, negout_hbm, wid * neg_per, neg_per // _CHUNK)
        run_pipelined(tgtidx_v, tgtout_hbm, wid * tgt_per, tgt_per // _CHUNK)

    return gather_k


def kernel(x):
    B, T, C = x.shape
    N = N_NEG
    key = jax.random.key(42)
    raw = jax.random.randint(key, (B, N * T), 0, T - 1)
    tszs = jnp.repeat(jnp.arange(T), N)
    loc = jnp.where(raw >= tszs[None, :], raw + 1, raw)
    g = loc + 1
    g = jnp.where(g >= T, g - T, g)
    neg_idx = g + jnp.arange(B)[:, None] * T
    neg_idx = neg_idx.reshape(B, T, N).transpose(2, 0, 1).reshape(-1)
    neg_idx = neg_idx.astype(jnp.int32)

    t = jnp.arange(T)
    tshift = jnp.where(t + 1 >= T, 0, t + 1)
    tgt_idx = (tshift[None, :] + jnp.arange(B)[:, None] * T).reshape(-1)
    tgt_idx = tgt_idx.astype(jnp.int32)

    table = x.reshape(B * T, C)
    neg_out, tgt_out = _make_gather(N * B * T, B * T, C)(table, neg_idx, tgt_idx)
    return (x, tgt_out.reshape(B, T, C), neg_out.reshape(N, B, T, C))


# speedup vs baseline: 7.0507x; 1.1707x over previous
import functools

import jax
import jax.numpy as jnp
import numpy as np
from jax import lax
from jax.experimental import pallas as pl
from jax.experimental.pallas import tpu as pltpu
from jax.experimental.pallas import tpu_sc as plsc

N_NEG = 10


def _tf2x32(k1, k2, x1, x2):
    rot_a = (13, 15, 26, 6)
    rot_b = (17, 29, 16, 24)

    def rotl(x, d):
        return ((x << np.uint32(d)) | (x >> np.uint32(32 - d))).astype(np.uint32)

    ks0 = np.uint32(k1)
    ks1 = np.uint32(k2)
    ks2 = np.uint32(ks0 ^ ks1 ^ np.uint32(0x1BD11BDA))
    x = [(x1 + ks0).astype(np.uint32), (x2 + ks1).astype(np.uint32)]

    def rounds(x, rots):
        for r in rots:
            x[0] = (x[0] + x[1]).astype(np.uint32)
            x[1] = (x[0] ^ rotl(x[1], r)).astype(np.uint32)
        return x

    ks = (ks0, ks1, ks2)
    for i, rots in enumerate((rot_a, rot_b, rot_a, rot_b, rot_a)):
        x = rounds(x, rots)
        x[0] = (x[0] + ks[(i + 1) % 3]).astype(np.uint32)
        x[1] = (x[1] + ks[(i + 2) % 3] + np.uint32(i + 1)).astype(np.uint32)
    return x[0], x[1]


def _iota2x32(shape):
    n = int(np.prod(shape))
    c = np.arange(n, dtype=np.uint64)
    return (
        (c >> np.uint64(32)).astype(np.uint32).reshape(shape),
        (c & np.uint64(0xFFFFFFFF)).astype(np.uint32).reshape(shape),
    )


def _np_randint(key, shape, minval, maxval):
    c1, c2 = _iota2x32((2,))
    b1, b2 = _tf2x32(key[0], key[1], c1, c2)
    subkeys = np.stack([b1, b2], axis=1)

    def random_bits(k):
        h1, h2 = _iota2x32(shape)
        r1, r2 = _tf2x32(k[0], k[1], h1, h2)
        return (r1 ^ r2).astype(np.uint32)

    hi, lo = random_bits(subkeys[0]), random_bits(subkeys[1])
    span = np.uint32(maxval - minval)
    mult = np.uint32((((2 ** 16) % int(span)) ** 2) % int(span))
    off = ((hi % span) * mult + (lo % span)).astype(np.uint32) % span
    return np.int32(minval) + off.astype(np.int32)


@functools.lru_cache(maxsize=None)
def _make_indices(B, T, N):
    key = np.array([0, 42], dtype=np.uint32)
    raw = _np_randint(key, (B, N * T), 0, T - 1)
    tszs = np.repeat(np.arange(T, dtype=np.int32), N)
    loc = np.where(raw >= tszs[None, :], raw + 1, raw)
    g = loc + 1
    g = np.where(g >= T, g - T, g)
    neg_idx = g + np.arange(B, dtype=np.int32)[:, None] * T
    neg_idx = np.ascontiguousarray(
        neg_idx.reshape(B, T, N).transpose(2, 0, 1)
    ).reshape(-1).astype(np.int32)

    t = np.arange(T, dtype=np.int32)
    tshift = np.where(t + 1 >= T, 0, t + 1)
    tgt_idx = (tshift[None, :] + np.arange(B, dtype=np.int32)[:, None] * T)
    tgt_idx = tgt_idx.reshape(-1).astype(np.int32)
    return neg_idx, tgt_idx

_NC = 2
_NS = 16
_NW = _NC * _NS
_CHUNK = 128


@functools.lru_cache(maxsize=None)
def _make_gather(n_neg_rows, n_tgt_rows, C):
    neg_per = n_neg_rows // _NW
    tgt_per = n_tgt_rows // _NW
    assert neg_per % _CHUNK == 0 and tgt_per % _CHUNK == 0

    mesh = plsc.VectorSubcoreMesh(core_axis_name="c", subcore_axis_name="s")

    @functools.partial(
        pl.kernel,
        mesh=mesh,
        out_type=[
            jax.ShapeDtypeStruct((n_neg_rows, C), jnp.float32),
            jax.ShapeDtypeStruct((n_tgt_rows, C), jnp.float32),
        ],
        scratch_types=[
            pltpu.VMEM((neg_per,), jnp.int32),
            pltpu.VMEM((tgt_per,), jnp.int32),
            pltpu.VMEM((_CHUNK, C), jnp.float32),
            pltpu.VMEM((_CHUNK, C), jnp.float32),
            pltpu.SemaphoreType.DMA,
            pltpu.SemaphoreType.DMA,
            pltpu.SemaphoreType.DMA,
            pltpu.SemaphoreType.DMA,
        ],
    )
    def gather_k(table_hbm, negidx_hbm, tgtidx_hbm, negout_hbm, tgtout_hbm,
                 negidx_v, tgtidx_v, rows0_v, rows1_v,
                 sem_g0, sem_g1, sem_s0, sem_s1):
        wid = lax.axis_index("s") * _NC + lax.axis_index("c")
        pltpu.sync_copy(negidx_hbm.at[pl.ds(wid * neg_per, neg_per)], negidx_v)
        pltpu.sync_copy(tgtidx_hbm.at[pl.ds(wid * tgt_per, tgt_per)], tgtidx_v)

        def run_pipelined(idx_v, out_hbm, out_base, n_chunks):
            def drain(rows_v, sem):
                pltpu.make_async_copy(
                    rows_v, out_hbm.at[pl.ds(0, _CHUNK)], sem
                ).wait()

            def body(i, carry):
                c0 = 2 * i * _CHUNK
                c1 = c0 + _CHUNK

                @pl.when(i > 0)
                def _():
                    drain(rows0_v, sem_s0)
                    drain(rows1_v, sem_s1)

                g0 = pltpu.async_copy(
                    table_hbm.at[idx_v.at[pl.ds(c0, _CHUNK)]], rows0_v, sem_g0
                )
                g1 = pltpu.async_copy(
                    table_hbm.at[idx_v.at[pl.ds(c1, _CHUNK)]], rows1_v, sem_g1
                )
                g0.wait()
                pltpu.async_copy(
                    rows0_v, out_hbm.at[pl.ds(out_base + c0, _CHUNK)], sem_s0
                )
                g1.wait()
                pltpu.async_copy(
                    rows1_v, out_hbm.at[pl.ds(out_base + c1, _CHUNK)], sem_s1
                )
                return carry

            lax.fori_loop(0, n_chunks // 2, body, 0)
            drain(rows0_v, sem_s0)
            drain(rows1_v, sem_s1)

        run_pipelined(negidx_v, negout_hbm, wid * neg_per, neg_per // _CHUNK)
        run_pipelined(tgtidx_v, tgtout_hbm, wid * tgt_per, tgt_per // _CHUNK)

    return gather_k


def kernel(x):
    B, T, C = x.shape
    N = N_NEG
    neg_idx, tgt_idx = _make_indices(B, T, N)
    table = x.reshape(B * T, C)
    neg_out, tgt_out = _make_gather(N * B * T, B * T, C)(
        table, jnp.asarray(neg_idx), jnp.asarray(tgt_idx)
    )
    return (x, tgt_out.reshape(B, T, C), neg_out.reshape(N, B, T, C))


# speedup vs baseline: 7.5099x; 1.0651x over previous
import functools

import jax
import jax.numpy as jnp
import numpy as np
from jax import lax
from jax.experimental import pallas as pl
from jax.experimental.pallas import tpu as pltpu
from jax.experimental.pallas import tpu_sc as plsc

N_NEG = 10

_NC = 2
_NS = 16
_NW = _NC * _NS
_CHUNK = 128
_NBUF = 3


def _tf2x32(k1, k2, x1, x2):
    rot_a = (13, 15, 26, 6)
    rot_b = (17, 29, 16, 24)

    def rotl(x, d):
        return ((x << np.uint32(d)) | (x >> np.uint32(32 - d))).astype(np.uint32)

    ks0 = np.uint32(k1)
    ks1 = np.uint32(k2)
    ks2 = np.uint32(ks0 ^ ks1 ^ np.uint32(0x1BD11BDA))
    x = [(x1 + ks0).astype(np.uint32), (x2 + ks1).astype(np.uint32)]

    def rounds(x, rots):
        for r in rots:
            x[0] = (x[0] + x[1]).astype(np.uint32)
            x[1] = (x[0] ^ rotl(x[1], r)).astype(np.uint32)
        return x

    ks = (ks0, ks1, ks2)
    for i, rots in enumerate((rot_a, rot_b, rot_a, rot_b, rot_a)):
        x = rounds(x, rots)
        x[0] = (x[0] + ks[(i + 1) % 3]).astype(np.uint32)
        x[1] = (x[1] + ks[(i + 2) % 3] + np.uint32(i + 1)).astype(np.uint32)
    return x[0], x[1]


def _iota2x32(shape):
    n = int(np.prod(shape))
    c = np.arange(n, dtype=np.uint64)
    return (
        (c >> np.uint64(32)).astype(np.uint32).reshape(shape),
        (c & np.uint64(0xFFFFFFFF)).astype(np.uint32).reshape(shape),
    )


def _np_randint(key, shape, minval, maxval):
    c1, c2 = _iota2x32((2,))
    b1, b2 = _tf2x32(key[0], key[1], c1, c2)
    subkeys = np.stack([b1, b2], axis=1)

    def random_bits(k):
        h1, h2 = _iota2x32(shape)
        r1, r2 = _tf2x32(k[0], k[1], h1, h2)
        return (r1 ^ r2).astype(np.uint32)

    hi, lo = random_bits(subkeys[0]), random_bits(subkeys[1])
    span = np.uint32(maxval - minval)
    mult = np.uint32((((2 ** 16) % int(span)) ** 2) % int(span))
    off = ((hi % span) * mult + (lo % span)).astype(np.uint32) % span
    return np.int32(minval) + off.astype(np.int32)


@functools.lru_cache(maxsize=None)
def _make_indices(B, T, N):
    key = np.array([0, 42], dtype=np.uint32)
    raw = _np_randint(key, (B, N * T), 0, T - 1)
    tszs = np.repeat(np.arange(T, dtype=np.int32), N)
    loc = np.where(raw >= tszs[None, :], raw + 1, raw)
    g = loc + 1
    g = np.where(g >= T, g - T, g)
    neg_idx = g + np.arange(B, dtype=np.int32)[:, None] * T
    neg_idx = np.ascontiguousarray(
        neg_idx.reshape(B, T, N).transpose(2, 0, 1)
    ).reshape(-1).astype(np.int32)
    return neg_idx


@functools.lru_cache(maxsize=None)
def _make_gather(n_rows, C):
    per_tile = n_rows // _NW
    n_chunks = per_tile // _CHUNK
    n_groups, n_tail = divmod(n_chunks, _NBUF)

    mesh = plsc.VectorSubcoreMesh(core_axis_name="c", subcore_axis_name="s")

    @functools.partial(
        pl.kernel,
        mesh=mesh,
        out_type=jax.ShapeDtypeStruct((n_rows, C), jnp.float32),
        scratch_types=[
            pltpu.VMEM((per_tile,), jnp.int32),
            pltpu.VMEM((_NBUF, _CHUNK, C), jnp.float32),
        ]
        + [pltpu.SemaphoreType.DMA] * (2 * _NBUF),
    )
    def gather_k(table_hbm, idx_hbm, out_hbm, idx_v, rows_v, *sems):
        sem_g = sems[:_NBUF]
        sem_s = sems[_NBUF:]
        wid = lax.axis_index("s") * _NC + lax.axis_index("c")
        out_base = wid * per_tile
        pltpu.sync_copy(idx_hbm.at[pl.ds(out_base, per_tile)], idx_v)

        def start_gather(c, j):
            return pltpu.async_copy(
                table_hbm.at[idx_v.at[pl.ds(c * _CHUNK, _CHUNK)]],
                rows_v.at[j],
                sem_g[j],
            )

        def start_scatter(c, j):
            return pltpu.async_copy(
                rows_v.at[j],
                out_hbm.at[pl.ds(out_base + c * _CHUNK, _CHUNK)],
                sem_s[j],
            )

        def drain_scatter(j):
            pltpu.make_async_copy(
                rows_v.at[j], out_hbm.at[pl.ds(0, _CHUNK)], sem_s[j]
            ).wait()

        def wait_gather(j):
            pltpu.make_async_copy(
                table_hbm.at[pl.ds(0, _CHUNK)], rows_v.at[j], sem_g[j]
            ).wait()

        def body(g, carry):
            c0 = g * _NBUF
            for j in range(_NBUF):
                @pl.when(g > 0)
                def _(j=j):
                    drain_scatter(j)
                start_gather(c0 + j, j)
            for j in range(_NBUF):
                wait_gather(j)
                start_scatter(c0 + j, j)
            return carry

        lax.fori_loop(0, n_groups, body, 0)
        for j in range(n_tail):
            drain_scatter(j)
            start_gather(n_groups * _NBUF + j, j)
        for j in range(n_tail):
            wait_gather(j)
            start_scatter(n_groups * _NBUF + j, j)
        for j in range(_NBUF):
            drain_scatter(j)

    return gather_k


@functools.lru_cache(maxsize=None)
def _make_roll(B, T, C):
    def roll_k(x_ref, out_ref):
        out_ref[0, : T - 1] = x_ref[0, 1:]
        out_ref[0, T - 1 :] = x_ref[0, :1]

    return pl.pallas_call(
        roll_k,
        grid=(B,),
        in_specs=[pl.BlockSpec((1, T, C), lambda b: (b, 0, 0))],
        out_specs=pl.BlockSpec((1, T, C), lambda b: (b, 0, 0)),
        out_shape=jax.ShapeDtypeStruct((B, T, C), jnp.float32),
    )


def kernel(x):
    B, T, C = x.shape
    N = N_NEG
    neg_idx = _make_indices(B, T, N)
    table = x.reshape(B * T, C)
    neg_out = _make_gather(N * B * T, C)(table, jnp.asarray(neg_idx))
    targets = _make_roll(B, T, C)(x)
    return (x, targets, neg_out.reshape(N, B, T, C))


# speedup vs baseline: 7.5910x; 1.0108x over previous
import functools

import jax
import jax.numpy as jnp
import numpy as np
from jax import lax
from jax.experimental import pallas as pl
from jax.experimental.pallas import tpu as pltpu
from jax.experimental.pallas import tpu_sc as plsc

N_NEG = 10

_NC = 2
_NS = 16
_NW = _NC * _NS
_CHUNK = 64
_NBUF = 6


def _tf2x32(k1, k2, x1, x2):
    rot_a = (13, 15, 26, 6)
    rot_b = (17, 29, 16, 24)

    def rotl(x, d):
        return ((x << np.uint32(d)) | (x >> np.uint32(32 - d))).astype(np.uint32)

    ks0 = np.uint32(k1)
    ks1 = np.uint32(k2)
    ks2 = np.uint32(ks0 ^ ks1 ^ np.uint32(0x1BD11BDA))
    x = [(x1 + ks0).astype(np.uint32), (x2 + ks1).astype(np.uint32)]

    def rounds(x, rots):
        for r in rots:
            x[0] = (x[0] + x[1]).astype(np.uint32)
            x[1] = (x[0] ^ rotl(x[1], r)).astype(np.uint32)
        return x

    ks = (ks0, ks1, ks2)
    for i, rots in enumerate((rot_a, rot_b, rot_a, rot_b, rot_a)):
        x = rounds(x, rots)
        x[0] = (x[0] + ks[(i + 1) % 3]).astype(np.uint32)
        x[1] = (x[1] + ks[(i + 2) % 3] + np.uint32(i + 1)).astype(np.uint32)
    return x[0], x[1]


def _iota2x32(shape):
    n = int(np.prod(shape))
    c = np.arange(n, dtype=np.uint64)
    return (
        (c >> np.uint64(32)).astype(np.uint32).reshape(shape),
        (c & np.uint64(0xFFFFFFFF)).astype(np.uint32).reshape(shape),
    )


def _np_randint(key, shape, minval, maxval):
    c1, c2 = _iota2x32((2,))
    b1, b2 = _tf2x32(key[0], key[1], c1, c2)
    subkeys = np.stack([b1, b2], axis=1)

    def random_bits(k):
        h1, h2 = _iota2x32(shape)
        r1, r2 = _tf2x32(k[0], k[1], h1, h2)
        return (r1 ^ r2).astype(np.uint32)

    hi, lo = random_bits(subkeys[0]), random_bits(subkeys[1])
    span = np.uint32(maxval - minval)
    mult = np.uint32((((2 ** 16) % int(span)) ** 2) % int(span))
    off = ((hi % span) * mult + (lo % span)).astype(np.uint32) % span
    return np.int32(minval) + off.astype(np.int32)


@functools.lru_cache(maxsize=None)
def _make_indices(B, T, N):
    key = np.array([0, 42], dtype=np.uint32)
    raw = _np_randint(key, (B, N * T), 0, T - 1)
    tszs = np.repeat(np.arange(T, dtype=np.int32), N)
    loc = np.where(raw >= tszs[None, :], raw + 1, raw)
    g = loc + 1
    g = np.where(g >= T, g - T, g)
    neg_idx = g + np.arange(B, dtype=np.int32)[:, None] * T
    neg_idx = np.ascontiguousarray(
        neg_idx.reshape(B, T, N).transpose(2, 0, 1)
    ).reshape(-1).astype(np.int32)
    return neg_idx


@functools.lru_cache(maxsize=None)
def _make_gather(n_rows, C):
    per_tile = n_rows // _NW
    n_chunks = per_tile // _CHUNK
    n_groups, n_tail = divmod(n_chunks, _NBUF)

    mesh = plsc.VectorSubcoreMesh(core_axis_name="c", subcore_axis_name="s")

    @functools.partial(
        pl.kernel,
        mesh=mesh,
        out_type=jax.ShapeDtypeStruct((n_rows, C), jnp.float32),
        scratch_types=[
            pltpu.VMEM((per_tile,), jnp.int32),
            pltpu.VMEM((_NBUF, _CHUNK, C), jnp.float32),
        ]
        + [pltpu.SemaphoreType.DMA] * (2 * _NBUF),
    )
    def gather_k(table_hbm, idx_hbm, out_hbm, idx_v, rows_v, *sems):
        sem_g = sems[:_NBUF]
        sem_s = sems[_NBUF:]
        wid = lax.axis_index("s") * _NC + lax.axis_index("c")
        out_base = wid * per_tile
        pltpu.sync_copy(idx_hbm.at[pl.ds(out_base, per_tile)], idx_v)

        def start_gather(c, j):
            return pltpu.async_copy(
                table_hbm.at[idx_v.at[pl.ds(c * _CHUNK, _CHUNK)]],
                rows_v.at[j],
                sem_g[j],
            )

        def start_scatter(c, j):
            return pltpu.async_copy(
                rows_v.at[j],
                out_hbm.at[pl.ds(out_base + c * _CHUNK, _CHUNK)],
                sem_s[j],
            )

        def drain_scatter(j):
            pltpu.make_async_copy(
                rows_v.at[j], out_hbm.at[pl.ds(0, _CHUNK)], sem_s[j]
            ).wait()

        def wait_gather(j):
            pltpu.make_async_copy(
                table_hbm.at[pl.ds(0, _CHUNK)], rows_v.at[j], sem_g[j]
            ).wait()

        def body(g, carry):
            c0 = g * _NBUF
            for j in range(_NBUF):
                @pl.when(g > 0)
                def _(j=j):
                    drain_scatter(j)
                start_gather(c0 + j, j)
            for j in range(_NBUF):
                wait_gather(j)
                start_scatter(c0 + j, j)
            return carry

        lax.fori_loop(0, n_groups, body, 0)
        for j in range(n_tail):
            drain_scatter(j)
            start_gather(n_groups * _NBUF + j, j)
        for j in range(n_tail):
            wait_gather(j)
            start_scatter(n_groups * _NBUF + j, j)
        for j in range(_NBUF):
            drain_scatter(j)

    return gather_k


@functools.lru_cache(maxsize=None)
def _make_roll(B, T, C):
    def roll_k(x_ref, out_ref):
        out_ref[0, : T - 1] = x_ref[0, 1:]
        out_ref[0, T - 1 :] = x_ref[0, :1]

    return pl.pallas_call(
        roll_k,
        grid=(B,),
        in_specs=[pl.BlockSpec((1, T, C), lambda b: (b, 0, 0))],
        out_specs=pl.BlockSpec((1, T, C), lambda b: (b, 0, 0)),
        out_shape=jax.ShapeDtypeStruct((B, T, C), jnp.float32),
    )


def kernel(x):
    B, T, C = x.shape
    N = N_NEG
    neg_idx = _make_indices(B, T, N)
    table = x.reshape(B * T, C)
    neg_out = _make_gather(N * B * T, C)(table, jnp.asarray(neg_idx))
    targets = _make_roll(B, T, C)(x)
    return (x, targets, neg_out.reshape(N, B, T, C))


# speedup vs baseline: 7.8137x; 1.0293x over previous
import functools

import jax
import jax.numpy as jnp
import numpy as np
from jax import lax
from jax.experimental import pallas as pl
from jax.experimental.pallas import tpu as pltpu
from jax.experimental.pallas import tpu_sc as plsc

N_NEG = 10

_NC = 2
_NS = 16
_NW = _NC * _NS
_CHUNK = 64
_NBUF = 6


def _tf2x32(k1, k2, x1, x2):
    rot_a = (13, 15, 26, 6)
    rot_b = (17, 29, 16, 24)

    def rotl(x, d):
        return ((x << np.uint32(d)) | (x >> np.uint32(32 - d))).astype(np.uint32)

    ks0 = np.uint32(k1)
    ks1 = np.uint32(k2)
    ks2 = np.uint32(ks0 ^ ks1 ^ np.uint32(0x1BD11BDA))
    x = [(x1 + ks0).astype(np.uint32), (x2 + ks1).astype(np.uint32)]

    def rounds(x, rots):
        for r in rots:
            x[0] = (x[0] + x[1]).astype(np.uint32)
            x[1] = (x[0] ^ rotl(x[1], r)).astype(np.uint32)
        return x

    ks = (ks0, ks1, ks2)
    for i, rots in enumerate((rot_a, rot_b, rot_a, rot_b, rot_a)):
        x = rounds(x, rots)
        x[0] = (x[0] + ks[(i + 1) % 3]).astype(np.uint32)
        x[1] = (x[1] + ks[(i + 2) % 3] + np.uint32(i + 1)).astype(np.uint32)
    return x[0], x[1]


def _iota2x32(shape):
    n = int(np.prod(shape))
    c = np.arange(n, dtype=np.uint64)
    return (
        (c >> np.uint64(32)).astype(np.uint32).reshape(shape),
        (c & np.uint64(0xFFFFFFFF)).astype(np.uint32).reshape(shape),
    )


def _np_randint(key, shape, minval, maxval):
    c1, c2 = _iota2x32((2,))
    b1, b2 = _tf2x32(key[0], key[1], c1, c2)
    subkeys = np.stack([b1, b2], axis=1)

    def random_bits(k):
        h1, h2 = _iota2x32(shape)
        r1, r2 = _tf2x32(k[0], k[1], h1, h2)
        return (r1 ^ r2).astype(np.uint32)

    hi, lo = random_bits(subkeys[0]), random_bits(subkeys[1])
    span = np.uint32(maxval - minval)
    mult = np.uint32((((2 ** 16) % int(span)) ** 2) % int(span))
    off = ((hi % span) * mult + (lo % span)).astype(np.uint32) % span
    return np.int32(minval) + off.astype(np.int32)


@functools.lru_cache(maxsize=None)
def _make_indices(B, T, N):
    key = np.array([0, 42], dtype=np.uint32)
    raw = _np_randint(key, (B, N * T), 0, T - 1)
    tszs = np.repeat(np.arange(T, dtype=np.int32), N)
    loc = np.where(raw >= tszs[None, :], raw + 1, raw)
    g = loc + 1
    g = np.where(g >= T, g - T, g)
    neg_idx = g + np.arange(B, dtype=np.int32)[:, None] * T
    neg_idx = np.ascontiguousarray(
        neg_idx.reshape(B, T, N).transpose(2, 0, 1)
    ).reshape(-1).astype(np.int32)
    return neg_idx


@functools.lru_cache(maxsize=None)
def _make_gather(n_rows, C):
    per_tile = n_rows // _NW
    n_chunks = per_tile // _CHUNK
    n_groups, n_tail = divmod(n_chunks, _NBUF)

    mesh = plsc.VectorSubcoreMesh(core_axis_name="c", subcore_axis_name="s")

    @functools.partial(
        pl.kernel,
        mesh=mesh,
        out_type=jax.ShapeDtypeStruct((n_rows, C), jnp.float32),
        scratch_types=[
            pltpu.VMEM((per_tile,), jnp.int32),
            pltpu.VMEM((_NBUF, _CHUNK, C), jnp.float32),
        ]
        + [pltpu.SemaphoreType.DMA] * (2 * _NBUF),
    )
    def gather_k(table_hbm, idx_hbm, out_hbm, idx_v, rows_v, *sems):
        sem_g = sems[:_NBUF]
        sem_s = sems[_NBUF:]
        wid = lax.axis_index("s") * _NC + lax.axis_index("c")
        out_base = wid * per_tile
        pltpu.sync_copy(idx_hbm.at[pl.ds(out_base, per_tile)], idx_v)

        def start_gather(c, j):
            return pltpu.async_copy(
                table_hbm.at[idx_v.at[pl.ds(c * _CHUNK, _CHUNK)]],
                rows_v.at[j],
                sem_g[j],
            )

        def start_scatter(c, j):
            return pltpu.async_copy(
                rows_v.at[j],
                out_hbm.at[pl.ds(out_base + c * _CHUNK, _CHUNK)],
                sem_s[j],
            )

        def drain_scatter(j):
            pltpu.make_async_copy(
                rows_v.at[j], out_hbm.at[pl.ds(0, _CHUNK)], sem_s[j]
            ).wait()

        def wait_gather(j):
            pltpu.make_async_copy(
                table_hbm.at[pl.ds(0, _CHUNK)], rows_v.at[j], sem_g[j]
            ).wait()

        def body(g, carry):
            c0 = g * _NBUF
            for j in range(_NBUF):
                @pl.when(g > 0)
                def _(j=j):
                    drain_scatter(j)
                start_gather(c0 + j, j)
            for j in range(_NBUF):
                wait_gather(j)
                start_scatter(c0 + j, j)
            return carry

        lax.fori_loop(0, n_groups, body, 0)
        for j in range(n_tail):
            drain_scatter(j)
            start_gather(n_groups * _NBUF + j, j)
        for j in range(n_tail):
            wait_gather(j)
            start_scatter(n_groups * _NBUF + j, j)
        for j in range(_NBUF):
            drain_scatter(j)

    return gather_k


@functools.lru_cache(maxsize=None)
def _make_roll(B, T, C):
    def roll_k(x_ref, out_ref, xcopy_ref):
        out_ref[0, : T - 1] = x_ref[0, 1:]
        out_ref[0, T - 1 :] = x_ref[0, :1]
        xcopy_ref[0] = x_ref[0]

    return pl.pallas_call(
        roll_k,
        grid=(B,),
        in_specs=[pl.BlockSpec((1, T, C), lambda b: (b, 0, 0))],
        out_specs=[
            pl.BlockSpec((1, T, C), lambda b: (b, 0, 0)),
            pl.BlockSpec((1, T, C), lambda b: (b, 0, 0)),
        ],
        out_shape=[
            jax.ShapeDtypeStruct((B, T, C), jnp.float32),
            jax.ShapeDtypeStruct((B, T, C), jnp.float32),
        ],
    )


def kernel(x):
    B, T, C = x.shape
    N = N_NEG
    neg_idx = _make_indices(B, T, N)
    table = x.reshape(B * T, C)
    neg_out = _make_gather(N * B * T, C)(table, jnp.asarray(neg_idx))
    targets, x_out = _make_roll(B, T, C)(x)
    return (x_out, targets, neg_out.reshape(N, B, T, C))


# speedup vs baseline: 7.8322x; 1.0024x over previous
import functools

import jax
import jax.numpy as jnp
import numpy as np
from jax import lax
from jax.experimental import pallas as pl
from jax.experimental.pallas import tpu as pltpu
from jax.experimental.pallas import tpu_sc as plsc

N_NEG = 10

_NC = 2
_NS = 16
_NW = _NC * _NS
_CHUNK = 128
_NBUF = 3


def _tf2x32(k1, k2, x1, x2):
    rot_a = (13, 15, 26, 6)
    rot_b = (17, 29, 16, 24)

    def rotl(x, d):
        return ((x << np.uint32(d)) | (x >> np.uint32(32 - d))).astype(np.uint32)

    ks0 = np.uint32(k1)
    ks1 = np.uint32(k2)
    ks2 = np.uint32(ks0 ^ ks1 ^ np.uint32(0x1BD11BDA))
    x = [(x1 + ks0).astype(np.uint32), (x2 + ks1).astype(np.uint32)]

    def rounds(x, rots):
        for r in rots:
            x[0] = (x[0] + x[1]).astype(np.uint32)
            x[1] = (x[0] ^ rotl(x[1], r)).astype(np.uint32)
        return x

    ks = (ks0, ks1, ks2)
    for i, rots in enumerate((rot_a, rot_b, rot_a, rot_b, rot_a)):
        x = rounds(x, rots)
        x[0] = (x[0] + ks[(i + 1) % 3]).astype(np.uint32)
        x[1] = (x[1] + ks[(i + 2) % 3] + np.uint32(i + 1)).astype(np.uint32)
    return x[0], x[1]


def _iota2x32(shape):
    n = int(np.prod(shape))
    c = np.arange(n, dtype=np.uint64)
    return (
        (c >> np.uint64(32)).astype(np.uint32).reshape(shape),
        (c & np.uint64(0xFFFFFFFF)).astype(np.uint32).reshape(shape),
    )


def _np_randint(key, shape, minval, maxval):
    c1, c2 = _iota2x32((2,))
    b1, b2 = _tf2x32(key[0], key[1], c1, c2)
    subkeys = np.stack([b1, b2], axis=1)

    def random_bits(k):
        h1, h2 = _iota2x32(shape)
        r1, r2 = _tf2x32(k[0], k[1], h1, h2)
        return (r1 ^ r2).astype(np.uint32)

    hi, lo = random_bits(subkeys[0]), random_bits(subkeys[1])
    span = np.uint32(maxval - minval)
    mult = np.uint32((((2 ** 16) % int(span)) ** 2) % int(span))
    off = ((hi % span) * mult + (lo % span)).astype(np.uint32) % span
    return np.int32(minval) + off.astype(np.int32)


@functools.lru_cache(maxsize=None)
def _make_indices(B, T, N):
    key = np.array([0, 42], dtype=np.uint32)
    raw = _np_randint(key, (B, N * T), 0, T - 1)
    tszs = np.repeat(np.arange(T, dtype=np.int32), N)
    loc = np.where(raw >= tszs[None, :], raw + 1, raw)
    g = loc + 1
    g = np.where(g >= T, g - T, g)
    neg_idx = g + np.arange(B, dtype=np.int32)[:, None] * T
    neg_idx = np.ascontiguousarray(
        neg_idx.reshape(B, T, N).transpose(2, 0, 1)
    ).reshape(-1).astype(np.int32)
    return neg_idx


@functools.lru_cache(maxsize=None)
def _make_gather(n_rows, C):
    per_tile = n_rows // _NW
    n_chunks = per_tile // _CHUNK
    n_groups, n_tail = divmod(n_chunks, _NBUF)

    mesh = plsc.VectorSubcoreMesh(core_axis_name="c", subcore_axis_name="s")

    @functools.partial(
        pl.kernel,
        mesh=mesh,
        out_type=jax.ShapeDtypeStruct((n_rows, C), jnp.float32),
        scratch_types=[
            pltpu.VMEM((n_chunks, _CHUNK), jnp.int32),
            pltpu.VMEM((_NBUF, _CHUNK, C), jnp.float32),
        ]
        + [pltpu.SemaphoreType.DMA] * (2 * _NBUF),
    )
    def gather_k(table_hbm, idx_hbm, out_hbm, idx_v, rows_v, *sems):
        sem_g = sems[:_NBUF]
        sem_s = sems[_NBUF:]
        wid = lax.axis_index("s") * _NC + lax.axis_index("c")
        out_base = wid * per_tile
        pltpu.sync_copy(idx_hbm.at[wid], idx_v)

        def start_gather(c, j):
            return pltpu.async_copy(
                table_hbm.at[idx_v.at[c]],
                rows_v.at[j],
                sem_g[j],
            )

        def start_scatter(c, j):
            return pltpu.async_copy(
                rows_v.at[j],
                out_hbm.at[pl.ds(out_base + c * _CHUNK, _CHUNK)],
                sem_s[j],
            )

        def drain_scatter(j):
            pltpu.make_async_copy(
                rows_v.at[j], out_hbm.at[pl.ds(0, _CHUNK)], sem_s[j]
            ).wait()

        def wait_gather(j):
            pltpu.make_async_copy(
                table_hbm.at[pl.ds(0, _CHUNK)], rows_v.at[j], sem_g[j]
            ).wait()

        def body(g, carry):
            c0 = g * _NBUF
            for j in range(_NBUF):
                @pl.when(g > 0)
                def _(j=j):
                    drain_scatter(j)
                start_gather(c0 + j, j)
            for j in range(_NBUF):
                wait_gather(j)
                start_scatter(c0 + j, j)
            return carry

        lax.fori_loop(0, n_groups, body, 0)
        for j in range(n_tail):
            drain_scatter(j)
            start_gather(n_groups * _NBUF + j, j)
        for j in range(n_tail):
            wait_gather(j)
            start_scatter(n_groups * _NBUF + j, j)
        for j in range(_NBUF):
            drain_scatter(j)

    return gather_k


@functools.lru_cache(maxsize=None)
def _make_roll(B, T, C):
    def roll_k(x_ref, out_ref, xcopy_ref):
        out_ref[0, : T - 1] = x_ref[0, 1:]
        out_ref[0, T - 1 :] = x_ref[0, :1]
        xcopy_ref[0] = x_ref[0]

    return pl.pallas_call(
        roll_k,
        grid=(B,),
        in_specs=[pl.BlockSpec((1, T, C), lambda b: (b, 0, 0))],
        out_specs=[
            pl.BlockSpec((1, T, C), lambda b: (b, 0, 0)),
            pl.BlockSpec((1, T, C), lambda b: (b, 0, 0)),
        ],
        out_shape=[
            jax.ShapeDtypeStruct((B, T, C), jnp.float32),
            jax.ShapeDtypeStruct((B, T, C), jnp.float32),
        ],
    )


def kernel(x):
    B, T, C = x.shape
    N = N_NEG
    neg_idx = _make_indices(B, T, N)
    n_rows = N * B * T
    neg_idx3 = neg_idx.reshape(_NW, n_rows // (_NW * _CHUNK), _CHUNK)
    table = x.reshape(B * T, C)
    neg_out = _make_gather(n_rows, C)(table, jnp.asarray(neg_idx3))
    targets, x_out = _make_roll(B, T, C)(x)
    return (x_out, targets, neg_out.reshape(N, B, T, C))
